# Initial kernel scaffold; baseline (speedup 1.0000x reference)
import jax
import jax.numpy as jnp
from jax.experimental import pallas as pl


def kernel(x, conv1_w, bn1_gamma, bn1_beta, bn1_mean, bn1_var, blk_conv1_w, blk_bn1_gamma, blk_bn1_beta, blk_bn1_mean, blk_bn1_var, blk_conv2_w, blk_bn2_gamma, blk_bn2_beta, blk_bn2_mean, blk_bn2_var, fc_w, fc_b):
    raise NotImplementedError("write your pallas kernel here")


# speedup vs baseline: 5.9452x; 5.9452x over previous
import functools

import jax
import jax.numpy as jnp
from jax.experimental import pallas as pl
from jax.experimental.pallas import tpu as pltpu

G = 4
YC = 32


def _fold_bn(w_hwio, gamma, beta, mean, var, eps=1e-5):
    scale = gamma / jnp.sqrt(var + eps)
    return w_hwio * scale, beta - mean * scale


def _packed_weights(w_hwio):
    cin, cout = w_hwio.shape[2], w_hwio.shape[3]
    w9 = w_hwio.reshape(9, cin, cout)
    z = jnp.zeros((9, G * cin, G * cout), w_hwio.dtype)
    for i in range(G):
        z = z.at[:, i * cin:(i + 1) * cin, i * cout:(i + 1) * cout].set(w9)
    return z.reshape(9 * G * cin, G * cout).astype(jnp.bfloat16)


def _net_kernel(xp_ref, w1_ref, b1_ref, w2_ref, b2_ref, w3_ref, b3_ref,
                fw_ref, fb_ref, o_ref, a1_ref, a2_ref, *, H, W):
    f32 = jnp.float32
    bf16 = jnp.bfloat16
    C = a1_ref.shape[2]
    cmid = fw_ref.shape[0]
    yc = min(YC, H)

    zrow = jnp.zeros((1, W + 2, C), bf16)
    zcol = jnp.zeros((H + 2, 1, C), bf16)
    for ref in (a1_ref, a2_ref):
        ref[0:1] = zrow
        ref[H + 1:H + 2] = zrow
        ref[:, 0:1, :] = zcol
        ref[:, W + 1:W + 2, :] = zcol

    def conv_chunk(x, yy, w_ref):
        cin = x.shape[2]
        pieces = []
        for ky in range(3):
            for kx in range(3):
                pieces.append(
                    x[yy + ky:yy + ky + yc, kx:kx + W, :].reshape(yc * W, cin))
        pcat = jnp.concatenate(pieces, axis=1)
        return jnp.dot(pcat, w_ref[...], preferred_element_type=f32)

    xin = xp_ref[0]
    for yy in range(0, H, yc):
        y = conv_chunk(xin, yy, w1_ref)
        y = jnp.maximum(y + b1_ref[...], 0.0).astype(bf16)
        a1_ref[1 + yy:1 + yy + yc, 1:1 + W, :] = y.reshape(yc, W, C)

    a1 = a1_ref[...]
    for yy in range(0, H, yc):
        y = conv_chunk(a1, yy, w2_ref)
        y = jnp.maximum(y + b2_ref[...], 0.0).astype(bf16)
        a2_ref[1 + yy:1 + yy + yc, 1:1 + W, :] = y.reshape(yc, W, C)

    a2 = a2_ref[...]
    psum = jnp.zeros((1, C), f32)
    for yy in range(0, H, yc):
        y = conv_chunk(a2, yy, w3_ref)
        y = jnp.maximum(y + b3_ref[...], 0.0)
        psum = psum + jnp.sum(y, axis=0, keepdims=True)
    pooled = psum * (1.0 / (H * W))

    fw = fw_ref[...]
    outs = [jnp.dot(pooled[:, i * cmid:(i + 1) * cmid], fw,
                    preferred_element_type=f32) + fb_ref[...]
            for i in range(G)]
    o_ref[0] = jnp.concatenate(outs, axis=0)


def kernel(x, conv1_w, bn1_gamma, bn1_beta, bn1_mean, bn1_var,
           blk_conv1_w, blk_bn1_gamma, blk_bn1_beta, blk_bn1_mean, blk_bn1_var,
           blk_conv2_w, blk_bn2_gamma, blk_bn2_beta, blk_bn2_mean, blk_bn2_var,
           fc_w, fc_b):
    B, Cin, H, W = x.shape
    nc = fc_w.shape[1]
    cmid = conv1_w.shape[3]
    C = G * cmid
    NG = B // G

    w1, b1 = _fold_bn(conv1_w, bn1_gamma, bn1_beta, bn1_mean, bn1_var)
    w2, b2 = _fold_bn(blk_conv1_w, blk_bn1_gamma, blk_bn1_beta,
                      blk_bn1_mean, blk_bn1_var)
    w3, b3 = _fold_bn(blk_conv2_w, blk_bn2_gamma, blk_bn2_beta,
                      blk_bn2_mean, blk_bn2_var)

    xg = jnp.transpose(x, (0, 2, 3, 1)).reshape(NG, G, H, W, Cin)
    xg = jnp.transpose(xg, (0, 2, 3, 1, 4)).reshape(NG, H, W, G * Cin)
    xp = jnp.pad(xg, ((0, 0), (1, 1), (1, 1), (0, 0))).astype(jnp.bfloat16)

    w1c = _packed_weights(w1)
    w2c = _packed_weights(w2)
    w3c = _packed_weights(w3)
    b1t = jnp.tile(b1, (G,)).reshape(1, C)
    b2t = jnp.tile(b2, (G,)).reshape(1, C)
    b3t = jnp.tile(b3, (G,)).reshape(1, C)
    fcb = fc_b.reshape(1, nc)

    kern = functools.partial(_net_kernel, H=H, W=W)
    out = pl.pallas_call(
        kern,
        out_shape=jax.ShapeDtypeStruct((NG, G, nc), jnp.float32),
        grid=(NG,),
        in_specs=[
            pl.BlockSpec((1, H + 2, W + 2, G * Cin), lambda i: (i, 0, 0, 0)),
            pl.BlockSpec(w1c.shape, lambda i: (0, 0)),
            pl.BlockSpec(b1t.shape, lambda i: (0, 0)),
            pl.BlockSpec(w2c.shape, lambda i: (0, 0)),
            pl.BlockSpec(b2t.shape, lambda i: (0, 0)),
            pl.BlockSpec(w3c.shape, lambda i: (0, 0)),
            pl.BlockSpec(b3t.shape, lambda i: (0, 0)),
            pl.BlockSpec(fc_w.shape, lambda i: (0, 0)),
            pl.BlockSpec(fcb.shape, lambda i: (0, 0)),
        ],
        out_specs=pl.BlockSpec((1, G, nc), lambda i: (i, 0, 0)),
        scratch_shapes=[
            pltpu.VMEM((H + 2, W + 2, C), jnp.bfloat16),
            pltpu.VMEM((H + 2, W + 2, C), jnp.bfloat16),
        ],
        compiler_params=pltpu.CompilerParams(
            dimension_semantics=("parallel",)),
    )(xp, w1c, b1t, w2c, b2t, w3c, b3t, fc_w, fcb)
    return out.reshape(B, nc)


# speedup vs baseline: 6.4165x; 1.0793x over previous
import functools

import jax
import jax.numpy as jnp
from jax.experimental import pallas as pl
from jax.experimental.pallas import tpu as pltpu

G = 4
YC = 8


def _fold_bn(w_hwio, gamma, beta, mean, var, eps=1e-5):
    scale = gamma / jnp.sqrt(var + eps)
    return w_hwio * scale, beta - mean * scale


def _packed_weights(w_hwio):
    cin, cout = w_hwio.shape[2], w_hwio.shape[3]
    w9 = w_hwio.reshape(9, cin, cout)
    z = jnp.zeros((9, G * cin, G * cout), w_hwio.dtype)
    for i in range(G):
        z = z.at[:, i * cin:(i + 1) * cin, i * cout:(i + 1) * cout].set(w9)
    return z.reshape(9 * G * cin, G * cout).astype(jnp.bfloat16)


def _net_kernel(x3_ref, w1_ref, b1_ref, w2_ref, b2_ref, w3_ref, b3_ref,
                fw_ref, fb_ref, o_ref, wa_ref, wb_ref, *, H, W):
    f32 = jnp.float32
    bf16 = jnp.bfloat16
    C = wa_ref.shape[1] // 3
    cmid = fw_ref.shape[0]
    HW = H * W
    R = (H + 2) * W
    yc = min(YC, H)
    rc = yc * W

    zh = jnp.zeros((W, 3 * C), bf16)
    zc = jnp.zeros((1, C), bf16)
    for ref in (wa_ref, wb_ref):
        ref[0:W] = zh
        ref[(H + 1) * W:R] = zh
        ref[W:W + 1, 0:C] = zc
        ref[R - W - 1:R - W, 2 * C:3 * C] = zc

    idx = jax.lax.broadcasted_iota(jnp.int32, (rc, 1), 0)
    m0 = (idx % W) != (W - 1)
    m2 = (idx % W) != 0
    zv = jnp.zeros((rc, C), bf16)

    def store_widened(ref, v, yy):
        base = (1 + yy) * W
        ref[base:base + rc, C:2 * C] = v
        ref[base + 1:base + 1 + rc, 0:C] = jnp.where(m0, v, zv)
        ref[base - 1:base - 1 + rc, 2 * C:3 * C] = jnp.where(m2, v, zv)

    def conv_chunk(src_ref, w_ref, yy):
        pcat = jnp.concatenate(
            [src_ref[(yy + ky) * W:(yy + ky) * W + rc, :] for ky in range(3)],
            axis=1)
        return jnp.dot(pcat, w_ref[...], preferred_element_type=f32)

    for yy in range(0, H, yc):
        p1 = jnp.concatenate(
            [x3_ref[0, (yy + ky) * W:(yy + ky) * W + rc, :]
             for ky in range(3)], axis=1)
        y = jnp.dot(p1, w1_ref[...], preferred_element_type=f32)
        y = jnp.maximum(y + b1_ref[...], 0.0).astype(bf16)
        store_widened(wa_ref, y, yy)

    for yy in range(0, H, yc):
        y = conv_chunk(wa_ref, w2_ref, yy)
        y = jnp.maximum(y + b2_ref[...], 0.0).astype(bf16)
        store_widened(wb_ref, y, yy)

    psum = jnp.zeros((1, C), f32)
    for yy in range(0, H, yc):
        y = conv_chunk(wb_ref, w3_ref, yy)
        y = jnp.maximum(y + b3_ref[...], 0.0)
        psum = psum + jnp.sum(y, axis=0, keepdims=True)
    pooled = psum * (1.0 / HW)

    fw = fw_ref[...]
    outs = [jnp.dot(pooled[:, i * cmid:(i + 1) * cmid], fw,
                    preferred_element_type=f32) + fb_ref[...]
            for i in range(G)]
    o_ref[0] = jnp.concatenate(outs, axis=0)


def kernel(x, conv1_w, bn1_gamma, bn1_beta, bn1_mean, bn1_var,
           blk_conv1_w, blk_bn1_gamma, blk_bn1_beta, blk_bn1_mean, blk_bn1_var,
           blk_conv2_w, blk_bn2_gamma, blk_bn2_beta, blk_bn2_mean, blk_bn2_var,
           fc_w, fc_b):
    B, Cin, H, W = x.shape
    nc = fc_w.shape[1]
    cmid = conv1_w.shape[3]
    C = G * cmid
    NG = B // G
    R = (H + 2) * W

    w1, b1 = _fold_bn(conv1_w, bn1_gamma, bn1_beta, bn1_mean, bn1_var)
    w2, b2 = _fold_bn(blk_conv1_w, blk_bn1_gamma, blk_bn1_beta,
                      blk_bn1_mean, blk_bn1_var)
    w3, b3 = _fold_bn(blk_conv2_w, blk_bn2_gamma, blk_bn2_beta,
                      blk_bn2_mean, blk_bn2_var)

    xg = jnp.transpose(x.reshape(NG, G, Cin, H, W), (0, 3, 4, 1, 2))
    xg = xg.reshape(NG, H, W, G * Cin)
    xpw = jnp.pad(xg, ((0, 0), (1, 1), (1, 1), (0, 0)))
    x3 = jnp.concatenate([xpw[:, :, kx:kx + W, :] for kx in range(3)],
                         axis=-1)
    x3 = x3.reshape(NG, R, 3 * G * Cin).astype(jnp.bfloat16)

    w1c = _packed_weights(w1)
    w2c = _packed_weights(w2)
    w3c = _packed_weights(w3)
    b1t = jnp.tile(b1, (G,)).reshape(1, C)
    b2t = jnp.tile(b2, (G,)).reshape(1, C)
    b3t = jnp.tile(b3, (G,)).reshape(1, C)
    fcb = fc_b.reshape(1, nc)

    kern = functools.partial(_net_kernel, H=H, W=W)
    out = pl.pallas_call(
        kern,
        out_shape=jax.ShapeDtypeStruct((NG, G, nc), jnp.float32),
        grid=(NG,),
        in_specs=[
            pl.BlockSpec((1, R, 3 * G * Cin), lambda i: (i, 0, 0)),
            pl.BlockSpec(w1c.shape, lambda i: (0, 0)),
            pl.BlockSpec(b1t.shape, lambda i: (0, 0)),
            pl.BlockSpec(w2c.shape, lambda i: (0, 0)),
            pl.BlockSpec(b2t.shape, lambda i: (0, 0)),
            pl.BlockSpec(w3c.shape, lambda i: (0, 0)),
            pl.BlockSpec(b3t.shape, lambda i: (0, 0)),
            pl.BlockSpec(fc_w.shape, lambda i: (0, 0)),
            pl.BlockSpec(fcb.shape, lambda i: (0, 0)),
        ],
        out_specs=pl.BlockSpec((1, G, nc), lambda i: (i, 0, 0)),
        scratch_shapes=[
            pltpu.VMEM((R, 3 * C), jnp.bfloat16),
            pltpu.VMEM((R, 3 * C), jnp.bfloat16),
        ],
        compiler_params=pltpu.CompilerParams(
            dimension_semantics=("parallel",)),
    )(x3, w1c, b1t, w2c, b2t, w3c, b3t, fc_w, fcb)
    return out.reshape(B, nc)


# speedup vs baseline: 6.4908x; 1.0116x over previous
import functools

import jax
import jax.numpy as jnp
from jax.experimental import pallas as pl
from jax.experimental.pallas import tpu as pltpu

G = 4
YC = 8


def _fold_bn(w_hwio, gamma, beta, mean, var, eps=1e-5):
    scale = gamma / jnp.sqrt(var + eps)
    return w_hwio * scale, beta - mean * scale


def _packed_weights(w_hwio):
    cin, cout = w_hwio.shape[2], w_hwio.shape[3]
    w9 = w_hwio.reshape(9, cin, cout)
    z = jnp.zeros((9, G * cin, G * cout), w_hwio.dtype)
    for i in range(G):
        z = z.at[:, i * cin:(i + 1) * cin, i * cout:(i + 1) * cout].set(w9)
    return z.reshape(9 * G * cin, G * cout).astype(jnp.bfloat16)


def _net_kernel(x3_ref, w1_ref, b1_ref, w2_ref, b2_ref, w3_ref, b3_ref,
                fw_ref, fb_ref, o_ref, wa_ref, wb_ref, *, H, W):
    f32 = jnp.float32
    bf16 = jnp.bfloat16
    C = wa_ref.shape[1] // 3
    cmid = fw_ref.shape[0]
    HW = H * W
    R = (H + 2) * W
    yc = min(YC, H)
    rc = yc * W

    zh = jnp.zeros((W, 3 * C), bf16)
    zc = jnp.zeros((1, C), bf16)
    for ref in (wa_ref, wb_ref):
        ref[0:W] = zh
        ref[(H + 1) * W:R] = zh
        ref[W:W + 1, 0:C] = zc
        ref[R - W - 1:R - W, 2 * C:3 * C] = zc

    idx = jax.lax.broadcasted_iota(jnp.int32, (rc, 1), 0)
    m0 = (idx % W) != (W - 1)
    m2 = (idx % W) != 0
    zv = jnp.zeros((rc, C), bf16)

    def store_widened(ref, v, yy):
        base = (1 + yy) * W
        ref[base:base + rc, C:2 * C] = v
        ref[base + 1:base + 1 + rc, 0:C] = jnp.where(m0, v, zv)
        ref[base - 1:base - 1 + rc, 2 * C:3 * C] = jnp.where(m2, v, zv)

    def conv_chunk(src_ref, w_ref, yy):
        pcat = jnp.concatenate(
            [src_ref[(yy + ky) * W:(yy + ky) * W + rc, :] for ky in range(3)],
            axis=1)
        return jnp.dot(pcat, w_ref[...], preferred_element_type=f32)

    def c1(yy):
        p1 = jnp.concatenate(
            [x3_ref[0, (yy + ky) * W:(yy + ky) * W + rc, :]
             for ky in range(3)], axis=1)
        y = jnp.dot(p1, w1_ref[...], preferred_element_type=f32)
        y = jnp.maximum(y + b1_ref[...], 0.0).astype(bf16)
        store_widened(wa_ref, y, yy)

    def c2(yy):
        y = conv_chunk(wa_ref, w2_ref, yy)
        y = jnp.maximum(y + b2_ref[...], 0.0).astype(bf16)
        store_widened(wb_ref, y, yy)

    psums = []

    def c3(yy):
        y = conv_chunk(wb_ref, w3_ref, yy)
        y = jnp.maximum(y + b3_ref[...], 0.0)
        psums.append(jnp.sum(y, axis=0, keepdims=True))

    nck = H // yc
    sched = []
    done1 = done2 = done3 = 0
    while done3 < nck:
        if done1 < nck and done1 < done2 + 2:
            sched.append((c1, done1)); done1 += 1
        elif done2 < nck and done2 + 1 < done1 and done2 < done3 + 2:
            sched.append((c2, done2)); done2 += 1
        elif done3 < nck and done3 + 1 < done2:
            sched.append((c3, done3)); done3 += 1
        elif done1 < nck:
            sched.append((c1, done1)); done1 += 1
        elif done2 < nck:
            sched.append((c2, done2)); done2 += 1
        else:
            sched.append((c3, done3)); done3 += 1
    for fn, k in sched:
        fn(k * yc)
    psum = psums[0]
    for p in psums[1:]:
        psum = psum + p
    pooled = psum * (1.0 / HW)

    fw = fw_ref[...]
    outs = [jnp.dot(pooled[:, i * cmid:(i + 1) * cmid], fw,
                    preferred_element_type=f32) + fb_ref[...]
            for i in range(G)]
    o_ref[0] = jnp.concatenate(outs, axis=0)


def kernel(x, conv1_w, bn1_gamma, bn1_beta, bn1_mean, bn1_var,
           blk_conv1_w, blk_bn1_gamma, blk_bn1_beta, blk_bn1_mean, blk_bn1_var,
           blk_conv2_w, blk_bn2_gamma, blk_bn2_beta, blk_bn2_mean, blk_bn2_var,
           fc_w, fc_b):
    B, Cin, H, W = x.shape
    nc = fc_w.shape[1]
    cmid = conv1_w.shape[3]
    C = G * cmid
    NG = B // G
    R = (H + 2) * W

    w1, b1 = _fold_bn(conv1_w, bn1_gamma, bn1_beta, bn1_mean, bn1_var)
    w2, b2 = _fold_bn(blk_conv1_w, blk_bn1_gamma, blk_bn1_beta,
                      blk_bn1_mean, blk_bn1_var)
    w3, b3 = _fold_bn(blk_conv2_w, blk_bn2_gamma, blk_bn2_beta,
                      blk_bn2_mean, blk_bn2_var)

    xb = x.astype(jnp.bfloat16)
    xg = jnp.transpose(xb.reshape(NG, G, Cin, H, W), (0, 3, 4, 1, 2))
    xg = xg.reshape(NG, H, W, G * Cin)
    xpw = jnp.pad(xg, ((0, 0), (1, 1), (1, 1), (0, 0)))
    x3 = jnp.concatenate([xpw[:, :, kx:kx + W, :] for kx in range(3)],
                         axis=-1)
    x3 = x3.reshape(NG, R, 3 * G * Cin)

    w1c = _packed_weights(w1)
    w2c = _packed_weights(w2)
    w3c = _packed_weights(w3)
    b1t = jnp.tile(b1, (G,)).reshape(1, C)
    b2t = jnp.tile(b2, (G,)).reshape(1, C)
    b3t = jnp.tile(b3, (G,)).reshape(1, C)
    fcb = fc_b.reshape(1, nc)

    kern = functools.partial(_net_kernel, H=H, W=W)
    out = pl.pallas_call(
        kern,
        out_shape=jax.ShapeDtypeStruct((NG, G, nc), jnp.float32),
        grid=(NG,),
        in_specs=[
            pl.BlockSpec((1, R, 3 * G * Cin), lambda i: (i, 0, 0)),
            pl.BlockSpec(w1c.shape, lambda i: (0, 0)),
            pl.BlockSpec(b1t.shape, lambda i: (0, 0)),
            pl.BlockSpec(w2c.shape, lambda i: (0, 0)),
            pl.BlockSpec(b2t.shape, lambda i: (0, 0)),
            pl.BlockSpec(w3c.shape, lambda i: (0, 0)),
            pl.BlockSpec(b3t.shape, lambda i: (0, 0)),
            pl.BlockSpec(fc_w.shape, lambda i: (0, 0)),
            pl.BlockSpec(fcb.shape, lambda i: (0, 0)),
        ],
        out_specs=pl.BlockSpec((1, G, nc), lambda i: (i, 0, 0)),
        scratch_shapes=[
            pltpu.VMEM((R, 3 * C), jnp.bfloat16),
            pltpu.VMEM((R, 3 * C), jnp.bfloat16),
        ],
        compiler_params=pltpu.CompilerParams(
            dimension_semantics=("parallel",)),
    )(x3, w1c, b1t, w2c, b2t, w3c, b3t, fc_w, fcb)
    return out.reshape(B, nc)
